# baseline (speedup 1.0000x reference)
import functools

import jax
import jax.numpy as jnp
from jax import lax
from jax.experimental import pallas as pl
from jax.experimental.pallas import tpu as pltpu
from jax.experimental.pallas import tpu_sc as plsc

B = 16384
D = 64
NC = 2
NS = 16
NW = NC * NS
BPW = B // NW
CHUNK = 128
NCHUNK = BPW // CHUNK


def _body(uid, iid, cid, bid, ut, it, ct, bt, out_hbm,
          idx0, idx1, idx2, idx3, rows0, rows1, rows2, rows3,
          sem0, sem1, sem2, sem3):
    wid = lax.axis_index("s") * NC + lax.axis_index("c")
    idx_hbms = (uid, iid, cid, bid)
    tables = (ut, it, ct, bt)
    idx_v = (idx0, idx1, idx2, idx3)
    rows_v = (rows0, rows1, rows2, rows3)
    sems = (sem0, sem1, sem2, sem3)

    def chunk(ci, carry):
        cbase = wid * BPW + ci * CHUNK
        copies = []
        for f in range(4):
            pltpu.sync_copy(idx_hbms[f].at[wid, ci], idx_v[f])
            copies.append(pltpu.async_copy(tables[f].at[idx_v[f]], rows_v[f], sems[f]))
        for f in range(4):
            copies[f].wait()
            pltpu.sync_copy(rows_v[f], out_hbm.at[pl.ds(cbase, CHUNK), f])
        return carry

    lax.fori_loop(0, NCHUNK, chunk, 0, unroll=False)


_sc_call = pl.kernel(
    _body,
    out_type=jax.ShapeDtypeStruct((B, 4, D), jnp.float32),
    mesh=plsc.VectorSubcoreMesh(core_axis_name="c", subcore_axis_name="s"),
    compiler_params=pltpu.CompilerParams(use_tc_tiling_on_sc=False),
    scratch_types=(
        [pltpu.VMEM((CHUNK,), jnp.int32) for _ in range(4)]
        + [pltpu.VMEM((CHUNK, D), jnp.float32) for _ in range(4)]
        + [pltpu.SemaphoreType.DMA for _ in range(4)]
    ),
)


def kernel(user_id, item_id, category, brand,
           user_table, item_table, category_table, brand_table):
    shp = (NW, NCHUNK, CHUNK)
    out = _sc_call(
        user_id.reshape(shp), item_id.reshape(shp),
        category.reshape(shp), brand.reshape(shp),
        user_table, item_table, category_table, brand_table,
    )
    return out.reshape(B, 4 * D)


# speedup vs baseline: 1.0008x; 1.0008x over previous
import functools

import jax
import jax.numpy as jnp
from jax import lax
from jax.experimental import pallas as pl
from jax.experimental.pallas import tpu as pltpu
from jax.experimental.pallas import tpu_sc as plsc

B = 16384
D = 64
NC = 2
NS = 16
NW = NC * NS
BPW = B // NW
CHUNK = 128
NCHUNK = BPW // CHUNK
NSLOT = 3


def _body(uid, iid, cid, bid, ut, it, ct, bt, out_hbm,
          idx_v, rows_v, gsem0, gsem1, gsem2, wsem0, wsem1, wsem2):
    wid = lax.axis_index("s") * NC + lax.axis_index("c")
    idx_hbms = (uid, iid, cid, bid)
    tables = (ut, it, ct, bt)
    gsems = (gsem0, gsem1, gsem2)
    wsems = (wsem0, wsem1, wsem2)

    for f in range(4):
        pltpu.sync_copy(idx_hbms[f].at[wid], idx_v.at[f])

    def fire(ci, slot):
        return [
            pltpu.async_copy(
                tables[f].at[idx_v.at[f, ci]],
                rows_v.at[slot, f],
                gsems[slot],
            )
            for f in range(4)
        ]

    def drain_and_write(ci, slot, gcopies):
        cbase = wid * BPW + ci * CHUNK
        ws = []
        for f in range(4):
            gcopies[f].wait()
            ws.append(
                pltpu.async_copy(
                    rows_v.at[slot, f],
                    out_hbm.at[pl.ds(cbase, CHUNK), f],
                    wsems[slot],
                )
            )
        return ws

    g = [fire(0, 0), fire(1, 1), fire(2, 2), None]
    w = [None] * NCHUNK
    w[0] = drain_and_write(0, 0, g[0])
    w[1] = drain_and_write(1, 1, g[1])
    for c in w[0]:
        c.wait()
    g[3] = fire(3, 0)
    w[2] = drain_and_write(2, 2, g[2])
    w[3] = drain_and_write(3, 0, g[3])
    for ci in range(1, NCHUNK):
        for c in w[ci]:
            c.wait()


_sc_call = pl.kernel(
    _body,
    out_type=jax.ShapeDtypeStruct((B, 4, D), jnp.float32),
    mesh=plsc.VectorSubcoreMesh(core_axis_name="c", subcore_axis_name="s"),
    compiler_params=pltpu.CompilerParams(use_tc_tiling_on_sc=False),
    scratch_types=(
        [
            pltpu.VMEM((4, NCHUNK, CHUNK), jnp.int32),
            pltpu.VMEM((NSLOT, 4, CHUNK, D), jnp.float32),
        ]
        + [pltpu.SemaphoreType.DMA for _ in range(6)]
    ),
)


def kernel(user_id, item_id, category, brand,
           user_table, item_table, category_table, brand_table):
    shp = (NW, NCHUNK, CHUNK)
    out = _sc_call(
        user_id.reshape(shp), item_id.reshape(shp),
        category.reshape(shp), brand.reshape(shp),
        user_table, item_table, category_table, brand_table,
    )
    return out.reshape(B, 4 * D)
